# Initial kernel scaffold; baseline (speedup 1.0000x reference)
import jax
import jax.numpy as jnp
from jax.experimental import pallas as pl


def kernel(x, W_router, b_router, experts_inter, experts_out):
    raise NotImplementedError("write your pallas kernel here")


# speedup vs baseline: 1.6962x; 1.6962x over previous
import functools
import jax
import jax.numpy as jnp
from jax import lax
from jax.experimental import pallas as pl
from jax.experimental.pallas import tpu as pltpu

NE = 8
NR = 7
CAP = 256
SEQ = 2048
HID = 1024
INTER = 2048
TILE = 256


def _router_body(x_ref, wr_ref, br_ref, gidx_ref, sw_ref, noopw_ref):
    x = x_ref[...]
    logits = lax.dot_general(
        x, wr_ref[...], (((1,), (1,)), ((), ())),
        preferred_element_type=jnp.float32) + br_ref[...]
    m = jnp.max(logits, axis=1, keepdims=True)
    ex = jnp.exp(logits - m)
    p = ex / jnp.sum(ex, axis=1, keepdims=True)

    colid = lax.broadcasted_iota(jnp.int32, (SEQ, NE), 1)
    m1 = jnp.max(p, axis=1, keepdims=True)
    j1 = jnp.min(jnp.where(p == m1, colid, NE), axis=1, keepdims=True)
    sel1 = colid == j1
    p2 = jnp.where(sel1, -jnp.inf, p)
    m2 = jnp.max(p2, axis=1, keepdims=True)
    j2 = jnp.min(jnp.where(p2 == m2, colid, NE), axis=1, keepdims=True)
    sel2 = colid == j2
    w8 = jnp.where(sel1 | sel2, p, 0.0)

    maskf = w8[:, :NR] > 0
    maskv = maskf.astype(jnp.float32)

    r = lax.broadcasted_iota(jnp.int32, (TILE, TILE), 0)
    c = lax.broadcasted_iota(jnp.int32, (TILE, TILE), 1)
    tril = (c <= r).astype(jnp.float32)
    run = jnp.zeros((1, NR), jnp.float32)
    pos_tiles = []
    for i in range(SEQ // TILE):
        t = maskv[i * TILE:(i + 1) * TILE, :]
        pt = lax.dot_general(tril, t, (((1,), (0,)), ((), ())),
                             precision=lax.Precision.HIGHEST,
                             preferred_element_type=jnp.float32) + run
        run = run + jnp.sum(t, axis=0, keepdims=True)
        pos_tiles.append(pt)
    pos = jnp.concatenate(pos_tiles, axis=0)
    keep = maskf & (pos <= CAP)
    slot = pos - 1.0

    rowid = lax.broadcasted_iota(jnp.int32, (SEQ, 1), 0).astype(jnp.float32)
    crange = lax.broadcasted_iota(jnp.int32, (SEQ, CAP), 1).astype(jnp.float32)
    for e in range(NR):
        oh = jnp.where((slot[:, e:e + 1] == crange) & keep[:, e:e + 1],
                       1.0, 0.0)
        tok = lax.dot_general(oh, rowid, (((0,), (0,)), ((), ())),
                              precision=lax.Precision.HIGHEST,
                              preferred_element_type=jnp.float32)
        cnt = lax.dot_general(oh, jnp.ones((SEQ, 1), jnp.float32),
                              (((0,), (0,)), ((), ())),
                              precision=lax.Precision.HIGHEST,
                              preferred_element_type=jnp.float32)
        swe = lax.dot_general(oh, w8[:, e:e + 1], (((0,), (0,)), ((), ())),
                              precision=lax.Precision.HIGHEST,
                              preferred_element_type=jnp.float32)
        gidx_ref[e, 0, :] = jnp.where(cnt[:, 0] > 0, tok[:, 0],
                                      float(SEQ)).astype(jnp.int32)
        sw_ref[e, 0, :] = swe[:, 0]
    gidx_ref[NR, 0, :] = jnp.full((CAP,), SEQ, jnp.int32)
    sw_ref[NR, 0, :] = jnp.zeros((CAP,), jnp.float32)
    noopw_ref[...] = w8[:, NR:NE]


def _ffn_body(x_ref, gidx_ref, sw_ref, noopw_ref, w1_ref, w2_ref, out_ref):
    e = pl.program_id(0)
    g = gidx_ref[0, 0, :]
    d = (lax.broadcasted_iota(jnp.int32, (CAP, SEQ), 1)
         == g[:, None]).astype(jnp.float32)
    xg = lax.dot_general(d, x_ref[...], (((1,), (0,)), ((), ())),
                         preferred_element_type=jnp.float32)
    h = lax.dot_general(xg, w1_ref[0], (((1,), (0,)), ((), ())),
                        preferred_element_type=jnp.float32)
    h = jnp.maximum(h, 0.0)
    y = lax.dot_general(h, w2_ref[0], (((1,), (0,)), ((), ())),
                        preferred_element_type=jnp.float32)
    y = y * sw_ref[0, 0, :][:, None]
    contrib = lax.dot_general(d, y, (((0,), (0,)), ((), ())),
                              preferred_element_type=jnp.float32)

    @pl.when(e == 0)
    def _():
        out_ref[...] = x_ref[...] * noopw_ref[...] + contrib

    @pl.when(e > 0)
    def _():
        out_ref[...] = out_ref[...] + contrib


def kernel(x, W_router, b_router, experts_inter, experts_out):
    B, S, H = x.shape
    xf = x.reshape(S, H)

    gidx, sw, noopw = pl.pallas_call(
        _router_body,
        out_shape=(
            jax.ShapeDtypeStruct((NE, 1, CAP), jnp.int32),
            jax.ShapeDtypeStruct((NE, 1, CAP), jnp.float32),
            jax.ShapeDtypeStruct((SEQ, 1), jnp.float32),
        ),
    )(xf, W_router, b_router.reshape(1, NE))

    out = pl.pallas_call(
        _ffn_body,
        grid=(NR,),
        in_specs=[
            pl.BlockSpec((SEQ, HID), lambda e: (0, 0)),
            pl.BlockSpec((1, 1, CAP), lambda e: (e, 0, 0)),
            pl.BlockSpec((1, 1, CAP), lambda e: (e, 0, 0)),
            pl.BlockSpec((SEQ, 1), lambda e: (0, 0)),
            pl.BlockSpec((1, HID, INTER), lambda e: (e, 0, 0)),
            pl.BlockSpec((1, INTER, HID), lambda e: (e, 0, 0)),
        ],
        out_specs=pl.BlockSpec((SEQ, HID), lambda e: (0, 0)),
        out_shape=jax.ShapeDtypeStruct((SEQ, HID), jnp.float32),
        compiler_params=pltpu.CompilerParams(
            dimension_semantics=("arbitrary",)),
    )(xf, gidx, sw, noopw, experts_inter, experts_out)

    return out.reshape(B, S, H)
